# baseline (speedup 1.0000x reference)
import functools

import jax
import jax.numpy as jnp
from jax import lax
from jax.experimental import pallas as pl
from jax.experimental.pallas import tpu as pltpu


def _pick_tile(n, target):
    t = min(target, max(8, -(-n // 8) * 8))
    t = (t // 8) * 8
    while t >= 8:
        if n % t == 0:
            return t
        t -= 8
    return target


def _seg_sum_kernel(batch_ref, x_ref, nb_ref, e_ref, ei_ref,
                    npart_ref, epart_ref,
                    iota_n_ref, lo_ref, hi_ref,
                    *, n_node_chunks, n_edge_chunks, chunks_per_core,
                    n_valid, e_valid, mask_n, mask_e, bchunk):
    c = pl.program_id(0)
    i = pl.program_id(1)
    chunk = c * chunks_per_core + i

    B = npart_ref.shape[0]
    TN = x_ref.shape[0]
    TE = e_ref.shape[0]

    @pl.when(i == 0)
    def _init():
        npart_ref[...] = jnp.zeros_like(npart_ref)
        epart_ref[...] = jnp.zeros_like(epart_ref)
        iota_n_ref[...] = lax.broadcasted_iota(jnp.int32, (B, TN), 0)
        N = batch_ref.shape[1]
        g = lax.broadcasted_iota(jnp.int32, (B, bchunk), 0)
        cnt = jnp.zeros((B, 1), jnp.float32)
        for s in range(N // bchunk):
            eq = batch_ref[0:1, s * bchunk:(s + 1) * bchunk] == g
            cnt += jnp.sum(eq.astype(jnp.float32), axis=1, keepdims=True)
        r = lax.broadcasted_iota(jnp.int32, (B, B), 0)
        cc = lax.broadcasted_iota(jnp.int32, (B, B), 1)
        tri = (cc < r).astype(jnp.float32)
        lo = jnp.dot(tri, cnt, preferred_element_type=jnp.float32,
                     precision=lax.Precision.HIGHEST)
        lo_ref[...] = lo.astype(jnp.int32)
        hi_ref[...] = (lo + cnt).astype(jnp.int32)

    dims = (((1,), (0,)), ((), ()))

    @pl.when(chunk < n_node_chunks)
    def _node():
        oh = nb_ref[...] == iota_n_ref[...]
        if mask_n:
            pos = chunk * TN + lax.broadcasted_iota(jnp.int32, (1, TN), 1)
            oh = jnp.logical_and(oh, pos < n_valid)
        npart_ref[...] += lax.dot_general(
            oh.astype(jnp.bfloat16), x_ref[...].astype(jnp.bfloat16),
            dims, preferred_element_type=jnp.float32)

    @pl.when(chunk < n_edge_chunks)
    def _edge():
        tgt = ei_ref[...]
        oh = jnp.logical_and(tgt >= lo_ref[...], tgt < hi_ref[...])
        if mask_e:
            pos = chunk * TE + lax.broadcasted_iota(jnp.int32, (1, TE), 1)
            oh = jnp.logical_and(oh, pos < e_valid)
        epart_ref[...] += lax.dot_general(
            oh.astype(jnp.bfloat16), e_ref[...].astype(jnp.bfloat16),
            dims, preferred_element_type=jnp.float32)


def _mlp_kernel(npart_ref, epart_ref, u_ref, w_ref, b_ref, out_ref):
    node_sum = jnp.sum(npart_ref[...], axis=0)
    edge_sum = jnp.sum(epart_ref[...], axis=0)
    feats = jnp.concatenate([node_sum, edge_sum, u_ref[...]], axis=1)
    y = jnp.dot(feats, w_ref[...], preferred_element_type=jnp.float32)
    out_ref[...] = jnp.maximum(y + b_ref[...], 0.0)


def kernel(x, edge_index, edge_attr, u, batch, W, b):
    N, Fx = x.shape
    E, Fe = edge_attr.shape
    B, Fu = u.shape
    out_dim = W.shape[1]

    tile_n = _pick_tile(N, 8192)
    tile_e = _pick_tile(E, 16384)

    batch = batch.astype(jnp.int32)
    nb = batch.reshape(1, N)
    ei1 = edge_index[1].astype(jnp.int32).reshape(1, E)
    bchunk = N
    for cand in (8192, 4096, 2048, 1024, 512, 256, 128):
        if N % cand == 0:
            bchunk = cand
            break

    n_node_chunks = pl.cdiv(N, tile_n)
    n_edge_chunks = pl.cdiv(E, tile_e)
    n_chunks = max(n_node_chunks, n_edge_chunks)
    num_cores = 2 if n_chunks > 1 else 1
    chunks_per_core = pl.cdiv(n_chunks, num_cores)

    node_map = lambda c, i: (jnp.minimum(c * chunks_per_core + i,
                                         n_node_chunks - 1), 0)
    edge_map = lambda c, i: (jnp.minimum(c * chunks_per_core + i,
                                         n_edge_chunks - 1), 0)
    nb_map = lambda c, i: (0, jnp.minimum(c * chunks_per_core + i,
                                          n_node_chunks - 1))
    eb_map = lambda c, i: (0, jnp.minimum(c * chunks_per_core + i,
                                          n_edge_chunks - 1))

    body = functools.partial(
        _seg_sum_kernel,
        n_node_chunks=n_node_chunks, n_edge_chunks=n_edge_chunks,
        chunks_per_core=chunks_per_core,
        n_valid=N, e_valid=E,
        mask_n=(N % tile_n != 0), mask_e=(E % tile_e != 0), bchunk=bchunk)

    vmem_need = (2 * 4 * (tile_n * (Fx + 1) + tile_e * (Fe + 1))
                 + 2 * (tile_n * Fx + tile_e * Fe)
                 + 4 * B * (tile_n + tile_e) + 3 * B * (tile_n + tile_e)
                 + 4 * num_cores * B * (Fx + Fe) + (4 << 20))
    vmem_limit = int(min(max(vmem_need, 32 << 20), 100 << 20))

    cost = pl.CostEstimate(
        flops=2 * B * (N * Fx + E * Fe),
        transcendentals=0,
        bytes_accessed=4 * ((Fx + 1) * N + (Fe + 1) * E
                            + num_cores * B * (Fx + Fe)),
    )

    npart, epart = pl.pallas_call(
        body,
        out_shape=(jax.ShapeDtypeStruct((num_cores, B, Fx), jnp.float32),
                   jax.ShapeDtypeStruct((num_cores, B, Fe), jnp.float32)),
        grid_spec=pltpu.PrefetchScalarGridSpec(
            num_scalar_prefetch=0,
            grid=(num_cores, chunks_per_core),
            in_specs=[
                pl.BlockSpec((1, N), lambda c, i: (0, 0)),
                pl.BlockSpec((tile_n, Fx), node_map),
                pl.BlockSpec((1, tile_n), nb_map),
                pl.BlockSpec((tile_e, Fe), edge_map),
                pl.BlockSpec((1, tile_e), eb_map),
            ],
            out_specs=[
                pl.BlockSpec((None, B, Fx), lambda c, i: (c, 0, 0)),
                pl.BlockSpec((None, B, Fe), lambda c, i: (c, 0, 0)),
            ],
            scratch_shapes=[
                pltpu.VMEM((B, tile_n), jnp.int32),
                pltpu.VMEM((B, 1), jnp.int32),
                pltpu.VMEM((B, 1), jnp.int32),
            ],
        ),
        compiler_params=pltpu.CompilerParams(
            dimension_semantics=("parallel", "arbitrary"),
            vmem_limit_bytes=vmem_limit),
        cost_estimate=cost,
    )(nb, x, nb, edge_attr, ei1)

    return pl.pallas_call(
        _mlp_kernel,
        out_shape=jax.ShapeDtypeStruct((B, out_dim), jnp.float32),
    )(npart, epart, u, W, b.reshape(1, out_dim))


# speedup vs baseline: 1.1104x; 1.1104x over previous
import functools

import jax
import jax.numpy as jnp
from jax import lax
from jax.experimental import pallas as pl
from jax.experimental.pallas import tpu as pltpu


def _pick_tile(n, target):
    t = min(target, max(8, -(-n // 8) * 8))
    t = (t // 8) * 8
    while t >= 8:
        if n % t == 0:
            return t
        t -= 8
    return target


def _seg_sum_kernel(x_ref, nb_ref, e_ref, ei_ref, lo_ref, hi_ref,
                    npart_ref, epart_ref,
                    iota_n_ref,
                    *, n_node_chunks, n_edge_chunks, chunks_per_core,
                    n_valid, e_valid, mask_n, mask_e):
    c = pl.program_id(0)
    i = pl.program_id(1)
    chunk = c * chunks_per_core + i

    B = npart_ref.shape[0]
    TN = x_ref.shape[0]
    TE = e_ref.shape[0]

    @pl.when(i == 0)
    def _init():
        npart_ref[...] = jnp.zeros_like(npart_ref)
        epart_ref[...] = jnp.zeros_like(epart_ref)
        iota_n_ref[...] = lax.broadcasted_iota(jnp.int32, (B, TN), 0)

    dims = (((1,), (0,)), ((), ()))

    @pl.when(chunk < n_node_chunks)
    def _node():
        oh = nb_ref[...] == iota_n_ref[...]
        if mask_n:
            pos = chunk * TN + lax.broadcasted_iota(jnp.int32, (1, TN), 1)
            oh = jnp.logical_and(oh, pos < n_valid)
        npart_ref[...] += lax.dot_general(
            oh.astype(jnp.bfloat16), x_ref[...].astype(jnp.bfloat16),
            dims, preferred_element_type=jnp.float32)

    @pl.when(chunk < n_edge_chunks)
    def _edge():
        tgt = ei_ref[...]
        oh = jnp.logical_and(tgt >= lo_ref[...], tgt < hi_ref[...])
        if mask_e:
            pos = chunk * TE + lax.broadcasted_iota(jnp.int32, (1, TE), 1)
            oh = jnp.logical_and(oh, pos < e_valid)
        epart_ref[...] += lax.dot_general(
            oh.astype(jnp.bfloat16), e_ref[...].astype(jnp.bfloat16),
            dims, preferred_element_type=jnp.float32)


def _bounds_kernel(nb_ref, lo_ref, hi_ref, *, chunk):
    B = lo_ref.shape[0]
    N = nb_ref.shape[1]
    g = lax.broadcasted_iota(jnp.int32, (B, chunk), 0)
    cnt = jnp.zeros((B, 1), jnp.float32)
    for s in range(N // chunk):
        eq = nb_ref[0:1, s * chunk:(s + 1) * chunk] == g
        cnt += jnp.sum(eq.astype(jnp.float32), axis=1, keepdims=True)
    r = lax.broadcasted_iota(jnp.int32, (B, B), 0)
    c = lax.broadcasted_iota(jnp.int32, (B, B), 1)
    tri = (c < r).astype(jnp.float32)
    lo = jnp.dot(tri, cnt, preferred_element_type=jnp.float32,
                 precision=lax.Precision.HIGHEST)
    lo_ref[...] = lo.astype(jnp.int32)
    hi_ref[...] = (lo + cnt).astype(jnp.int32)


def _mlp_kernel(npart_ref, epart_ref, u_ref, w_ref, b_ref, out_ref):
    node_sum = jnp.sum(npart_ref[...], axis=0)
    edge_sum = jnp.sum(epart_ref[...], axis=0)
    feats = jnp.concatenate([node_sum, edge_sum, u_ref[...]], axis=1)
    y = jnp.dot(feats, w_ref[...], preferred_element_type=jnp.float32)
    out_ref[...] = jnp.maximum(y + b_ref[...], 0.0)


def kernel(x, edge_index, edge_attr, u, batch, W, b):
    N, Fx = x.shape
    E, Fe = edge_attr.shape
    B, Fu = u.shape
    out_dim = W.shape[1]

    tile_n = _pick_tile(N, 8192)
    tile_e = _pick_tile(E, 16384)

    batch = batch.astype(jnp.int32)
    nb = batch.reshape(1, N)
    ei1 = edge_index[1].astype(jnp.int32).reshape(1, E)
    bchunk = N
    for cand in (8192, 4096, 2048, 1024, 512, 256, 128):
        if N % cand == 0:
            bchunk = cand
            break
    lo, hi = pl.pallas_call(
        functools.partial(_bounds_kernel, chunk=bchunk),
        out_shape=(jax.ShapeDtypeStruct((B, 1), jnp.int32),
                   jax.ShapeDtypeStruct((B, 1), jnp.int32)),
    )(nb)

    n_node_chunks = pl.cdiv(N, tile_n)
    n_edge_chunks = pl.cdiv(E, tile_e)
    n_chunks = max(n_node_chunks, n_edge_chunks)
    num_cores = 2 if n_chunks > 1 else 1
    chunks_per_core = pl.cdiv(n_chunks, num_cores)

    node_map = lambda c, i: (jnp.minimum(c * chunks_per_core + i,
                                         n_node_chunks - 1), 0)
    edge_map = lambda c, i: (jnp.minimum(c * chunks_per_core + i,
                                         n_edge_chunks - 1), 0)
    nb_map = lambda c, i: (0, jnp.minimum(c * chunks_per_core + i,
                                          n_node_chunks - 1))
    eb_map = lambda c, i: (0, jnp.minimum(c * chunks_per_core + i,
                                          n_edge_chunks - 1))

    body = functools.partial(
        _seg_sum_kernel,
        n_node_chunks=n_node_chunks, n_edge_chunks=n_edge_chunks,
        chunks_per_core=chunks_per_core,
        n_valid=N, e_valid=E,
        mask_n=(N % tile_n != 0), mask_e=(E % tile_e != 0))

    vmem_need = (2 * 4 * (tile_n * (Fx + 1) + tile_e * (Fe + 1))
                 + 2 * (tile_n * Fx + tile_e * Fe)
                 + 4 * B * (tile_n + tile_e) + 3 * B * (tile_n + tile_e)
                 + 4 * num_cores * B * (Fx + Fe) + (4 << 20))
    vmem_limit = int(min(max(vmem_need, 32 << 20), 100 << 20))

    cost = pl.CostEstimate(
        flops=2 * B * (N * Fx + E * Fe),
        transcendentals=0,
        bytes_accessed=4 * ((Fx + 1) * N + (Fe + 1) * E
                            + num_cores * B * (Fx + Fe)),
    )

    npart, epart = pl.pallas_call(
        body,
        out_shape=(jax.ShapeDtypeStruct((num_cores, B, Fx), jnp.float32),
                   jax.ShapeDtypeStruct((num_cores, B, Fe), jnp.float32)),
        grid_spec=pltpu.PrefetchScalarGridSpec(
            num_scalar_prefetch=0,
            grid=(num_cores, chunks_per_core),
            in_specs=[
                pl.BlockSpec((tile_n, Fx), node_map),
                pl.BlockSpec((1, tile_n), nb_map),
                pl.BlockSpec((tile_e, Fe), edge_map),
                pl.BlockSpec((1, tile_e), eb_map),
                pl.BlockSpec((B, 1), lambda c, i: (0, 0)),
                pl.BlockSpec((B, 1), lambda c, i: (0, 0)),
            ],
            out_specs=[
                pl.BlockSpec((None, B, Fx), lambda c, i: (c, 0, 0)),
                pl.BlockSpec((None, B, Fe), lambda c, i: (c, 0, 0)),
            ],
            scratch_shapes=[
                pltpu.VMEM((B, tile_n), jnp.int32),
            ],
        ),
        compiler_params=pltpu.CompilerParams(
            dimension_semantics=("parallel", "arbitrary"),
            vmem_limit_bytes=vmem_limit),
        cost_estimate=cost,
    )(x, nb, edge_attr, ei1, lo, hi)

    return pl.pallas_call(
        _mlp_kernel,
        out_shape=jax.ShapeDtypeStruct((B, out_dim), jnp.float32),
    )(npart, epart, u, W, b.reshape(1, out_dim))
